# Initial kernel scaffold; baseline (speedup 1.0000x reference)
import jax
import jax.numpy as jnp
from jax.experimental import pallas as pl


def kernel(command_logits, unified_args_logits, commands, args_tokens, args_mask):
    raise NotImplementedError("write your pallas kernel here")


# speedup vs baseline: 6.5570x; 6.5570x over previous
import math

import jax
import jax.numpy as jnp
from jax.experimental import pallas as pl
from jax.experimental.pallas import tpu as pltpu

_B, _S, _NC, _NA, _V = 16, 128, 6, 16, 512
_EOS = 3
_TOL = 3
_ALPHA = 2.0
_M = _B * _S * _NA
_BLK = 1024
_GRID = _M // _BLK
_SHIFT_W = [math.exp(-_ALPHA * abs(s)) for s in range(-_TOL, _TOL + 1)]
_W_TOT = sum(_SHIFT_W)


def _prep_body(clT_ref, cmds_ref, am_ref, wm_ref, cnum_ref, cden_ref):
    cmds = cmds_ref[...]
    eos = (cmds == _EOS).astype(jnp.float32)
    r = jax.lax.broadcasted_iota(jnp.int32, (_S, _S), 0)
    c = jax.lax.broadcasted_iota(jnp.int32, (_S, _S), 1)
    lower = (r <= c).astype(jnp.float32)
    cum = jnp.dot(eos, lower, preferred_element_type=jnp.float32)
    valid = (cum <= 1.0).astype(jnp.float32)

    x0 = clT_ref[0]
    m = x0
    for ci in range(1, _NC):
        m = jnp.maximum(m, clT_ref[ci])
    ssum = jnp.zeros_like(m)
    xt = jnp.zeros_like(m)
    for ci in range(_NC):
        xc = clT_ref[ci]
        ssum = ssum + jnp.exp(xc - m)
        xt = xt + jnp.where(cmds == ci, xc, 0.0)
    lse = m + jnp.log(ssum)
    closs = lse - xt
    closs = jnp.where(jnp.isnan(closs), 0.0, closs)
    cnum_ref[0, 0] = jnp.sum(closs * valid)
    cden_ref[0, 0] = jnp.sum(valid)

    for a in range(_NA):
        acc = jnp.zeros((_B, _S), jnp.float32)
        for ci in range(_NC):
            acc = acc + jnp.where(cmds == ci, am_ref[ci, a], 0.0)
        wm_ref[:, a, :] = acc * valid


def _args_body(x_ref, tok_ref, wm_ref, num_ref, den_ref):
    @pl.when(pl.program_id(0) == 0)
    def _init():
        num_ref[0, 0] = jnp.float32(0.0)
        den_ref[0, 0] = jnp.float32(0.0)

    x = x_ref[...]
    m = jnp.max(x, axis=1, keepdims=True)
    e = jnp.exp(x - m)
    ssum = jnp.sum(e, axis=1, keepdims=True)
    lse = m + jnp.log(ssum)

    tok = jnp.clip(tok_ref[...], 0, _V - 1)
    lane = jax.lax.broadcasted_iota(jnp.int32, (_BLK, _V), 1)
    acc = jnp.zeros((_BLK, _V), jnp.float32)
    for k, s in enumerate(range(-_TOL, _TOL + 1)):
        idx = jnp.clip(tok + s, 0, _V - 1)
        acc = acc + jnp.where(lane == idx, jnp.float32(_SHIFT_W[k]), 0.0)
    g = jnp.sum(acc * x, axis=1, keepdims=True)

    loss = (jnp.float32(_W_TOT) * lse - g) * jnp.float32(1.0 / (_W_TOT + 1e-8))
    loss = jnp.where(jnp.isnan(loss), 0.0, loss)
    wm = wm_ref[...]
    num_ref[0, 0] += jnp.sum(loss * wm)
    den_ref[0, 0] += jnp.sum(wm)


def kernel(command_logits, unified_args_logits, commands, args_tokens, args_mask):
    clT = command_logits.astype(jnp.float32).transpose(2, 0, 1)
    cmds = commands.astype(jnp.int32)

    wm_bas, cnum, cden = pl.pallas_call(
        _prep_body,
        out_shape=(
            jax.ShapeDtypeStruct((_B, _NA, _S), jnp.float32),
            jax.ShapeDtypeStruct((1, 1), jnp.float32),
            jax.ShapeDtypeStruct((1, 1), jnp.float32),
        ),
        in_specs=[
            pl.BlockSpec(memory_space=pltpu.VMEM),
            pl.BlockSpec(memory_space=pltpu.VMEM),
            pl.BlockSpec(memory_space=pltpu.SMEM),
        ],
        out_specs=(
            pl.BlockSpec(memory_space=pltpu.VMEM),
            pl.BlockSpec(memory_space=pltpu.SMEM),
            pl.BlockSpec(memory_space=pltpu.SMEM),
        ),
    )(clT, cmds, args_mask.astype(jnp.float32))

    wm_col = wm_bas.transpose(0, 2, 1).reshape(_M, 1)
    x2 = unified_args_logits.astype(jnp.float32).reshape(_M, _V)
    tok_col = args_tokens.astype(jnp.int32).reshape(_M, 1)

    num, den = pl.pallas_call(
        _args_body,
        grid=(_GRID,),
        out_shape=(
            jax.ShapeDtypeStruct((1, 1), jnp.float32),
            jax.ShapeDtypeStruct((1, 1), jnp.float32),
        ),
        in_specs=[
            pl.BlockSpec((_BLK, _V), lambda i: (i, 0)),
            pl.BlockSpec((_BLK, 1), lambda i: (i, 0)),
            pl.BlockSpec((_BLK, 1), lambda i: (i, 0)),
        ],
        out_specs=(
            pl.BlockSpec((1, 1), lambda i: (0, 0), memory_space=pltpu.SMEM),
            pl.BlockSpec((1, 1), lambda i: (0, 0), memory_space=pltpu.SMEM),
        ),
    )(x2, tok_col, wm_col)

    loss_cmd = cnum[0, 0] / (cden[0, 0] + 1e-8)
    den_s = den[0, 0]
    la = num[0, 0] / (den_s + 1e-8)
    loss_args = jnp.where(den_s < 1.0, jnp.float32(0.0), la)
    total = loss_cmd + loss_args
    return total, loss_cmd, loss_args
